# baseline (speedup 1.0000x reference)
import functools

import jax
import jax.numpy as jnp
from jax import lax
from jax.experimental import pallas as pl
from jax.experimental.pallas import tpu as pltpu
from jax.experimental.pallas import tpu_sc as plsc

BATCH = 8
NA = 20000
PRE = 6000
NOUT = 1000
THR = 0.7
LANES = 16
LCHUNK = NA // LANES
UNROLL = 5
NBINS = 1024
KMAX = 0x3F7FFFFF
GCH = 128
NGC = 48
PREP = NGC * GCH
HROWS = 24
SELPAD = 1024

_f32 = jnp.float32
_i32 = jnp.int32


def _sc_body(scores_hbm, planes_hbm, out_hbm,
             ord_a, ord_b, cnt_v, idx0, idx1, gbuf,
             sel, outbuf, sem0, sem1):
    wid = lax.axis_index("s") * 2 + lax.axis_index("c")

    @pl.when(wid < BATCH)
    def _():
        b = wid
        lanes = lax.iota(_i32, LANES)

        pltpu.sync_copy(scores_hbm.at[pl.ds(b * NA, NA)], gbuf.at[pl.ds(0, NA)])

        def key_of(sv):
            bits = plsc.bitcast(sv, _i32)
            return KMAX - bits

        def do_pass(p, ord_src, ord_dst, last):
            shift = 10 * p

            zv16 = jnp.zeros((LANES,), _i32)

            def zero_body(t, _):
                for u in range(4):
                    cnt_v[pl.ds((t * 4 + u) * LANES, LANES)] = zv16
                return 0
            lax.fori_loop(0, NBINS // 4, zero_body, 0)

            def elem_digit(t):
                pos = lanes * LCHUNK + t
                if ord_src is None:
                    ev = pos
                    kp = key_of(plsc.load_gather(gbuf, [ev]))
                    digit = kp & (NBINS - 1)
                else:
                    ov = plsc.load_gather(ord_src, [pos])
                    digit = lax.shift_right_logical(ov, 15) & (NBINS - 1)
                    ev = ov & 0x7FFF
                    kp = None
                return ev, digit, kp

            ones = jnp.ones((LANES,), _i32)

            @functools.partial(plsc.parallel_loop, 0, LCHUNK // UNROLL,
                               unroll=2)
            def _hist(t):
                for u in range(UNROLL):
                    _, digit, _ = elem_digit(t * UNROLL + u)
                    plsc.addupdate_scatter(cnt_v, [digit * LANES + lanes],
                                           ones)

            def scan_body(t, carry):
                for u in range(4):
                    sl = pl.ds((t * 4 + u) * LANES, LANES)
                    v = cnt_v[sl]
                    incl = plsc.cumsum(v)
                    tot = jnp.sum(v)
                    cnt_v[sl] = incl - v + carry
                    carry = carry + tot
                return carry
            lax.fori_loop(0, NBINS // 4, scan_body, jnp.int32(0))

            def perm_body(t, _):
                for u in range(UNROLL):
                    ev, digit, kp = elem_digit(t * UNROLL + u)
                    slot = digit * LANES + lanes
                    pos = plsc.load_gather(cnt_v, [slot])
                    if last:
                        val = ev
                    else:
                        if kp is None:
                            kp = key_of(plsc.load_gather(gbuf, [ev]))
                        nd = lax.shift_right_logical(kp, shift + 10) \
                            & (NBINS - 1)
                        val = ev | lax.shift_left(nd, 15)
                    plsc.store_scatter(ord_dst, [pos], val)
                    plsc.addupdate_scatter(cnt_v, [slot], ones)
                return 0
            lax.fori_loop(0, LCHUNK // UNROLL, perm_body, 0)

        do_pass(0, None, ord_b, False)
        do_pass(1, ord_b, ord_a, False)
        do_pass(2, ord_a, ord_b, True)

        HHALF = HROWS * GCH

        def fill_rel(ib, h):
            def fb(t, _):
                ib[pl.ds(t * LANES, LANES)] = \
                    ord_b[pl.ds(h * HHALF + t * LANES, LANES)]
                return 0
            lax.fori_loop(0, HHALF // LANES, fb, 0)

        descs = []
        for h, ib, sem in ((0, idx0, sem0), (1, idx1, sem1)):
            fill_rel(ib, h)
            for j in range(8):
                d = pltpu.make_async_copy(
                    planes_hbm.at[pl.ds((b * 8 + j) * NA, NA)].at[ib],
                    gbuf.at[pl.ds(j * PREP + h * HHALF, HHALF)],
                    sem)
                d.start()
                descs.append(d)

        def boxes_half(h):
            def boxes_t(t, _):
                sl = pl.ds(h * HHALF + t * LANES, LANES)
                ay1 = gbuf[pl.ds(0 * PREP + h * HHALF + t * LANES, LANES)]
                ax1 = gbuf[pl.ds(1 * PREP + h * HHALF + t * LANES, LANES)]
                ay2 = gbuf[pl.ds(2 * PREP + h * HHALF + t * LANES, LANES)]
                ax2 = gbuf[pl.ds(3 * PREP + h * HHALF + t * LANES, LANES)]
                d0 = gbuf[pl.ds(4 * PREP + h * HHALF + t * LANES, LANES)] \
                    * _f32(0.1)
                d1 = gbuf[pl.ds(5 * PREP + h * HHALF + t * LANES, LANES)] \
                    * _f32(0.1)
                d2 = gbuf[pl.ds(6 * PREP + h * HHALF + t * LANES, LANES)] \
                    * _f32(0.2)
                d3 = gbuf[pl.ds(7 * PREP + h * HHALF + t * LANES, LANES)] \
                    * _f32(0.2)
                hh = ay2 - ay1
                w = ax2 - ax1
                cy = ay1 + _f32(0.5) * hh + d0 * hh
                cx = ax1 + _f32(0.5) * w + d1 * w
                h2 = hh * jnp.exp(d2)
                w2 = w * jnp.exp(d3)
                y1 = cy - _f32(0.5) * h2
                x1 = cx - _f32(0.5) * w2
                y2 = y1 + h2
                x2 = x1 + w2
                one = _f32(1.0)
                zero = _f32(0.0)
                y1 = jnp.minimum(jnp.maximum(y1, zero), one)
                x1 = jnp.minimum(jnp.maximum(x1, zero), one)
                y2 = jnp.minimum(jnp.maximum(y2, zero), one)
                x2 = jnp.minimum(jnp.maximum(x2, zero), one)
                gbuf[pl.ds(0 * PREP + h * HHALF + t * LANES, LANES)] = y1
                gbuf[pl.ds(1 * PREP + h * HHALF + t * LANES, LANES)] = x1
                gbuf[pl.ds(2 * PREP + h * HHALF + t * LANES, LANES)] = y2
                gbuf[pl.ds(3 * PREP + h * HHALF + t * LANES, LANES)] = x2
                gbuf[pl.ds(4 * PREP + h * HHALF + t * LANES, LANES)] = \
                    (y2 - y1) * (x2 - x1)
                return 0
            lax.fori_loop(0, HHALF // LANES, boxes_t, 0)

        def sent_body(t, _):
            two = jnp.full((LANES,), 2.0, _f32)
            zv = jnp.zeros((LANES,), _f32)
            for r in range(4):
                sel[pl.ds(r * SELPAD + t * LANES, LANES)] = two
            sel[pl.ds(4 * SELPAD + t * LANES, LANES)] = zv
            return 0
        lax.fori_loop(0, SELPAD // LANES, sent_body, 0)

        def zout_body(t, _):
            outbuf[pl.ds(t * LANES, LANES)] = jnp.zeros((LANES,), _f32)
            return 0
        lax.fori_loop(0, (4 * NOUT) // LANES, zout_body, 0)

        thr = _f32(THR)
        eps = _f32(1e-12)

        BLK = 8

        def nms_cond_blk(limit):
            def cond(state):
                bi, nsel = state
                return jnp.logical_and(bi < limit // BLK, nsel < NOUT)
            return cond

        def nms_blk_body(state):
            bi, nsel = state
            i0 = bi * BLK
            vy1 = gbuf[pl.ds(0 * PREP + i0, BLK * 2)]
            vx1 = gbuf[pl.ds(1 * PREP + i0, BLK * 2)]
            vy2 = gbuf[pl.ds(2 * PREP + i0, BLK * 2)]
            vx2 = gbuf[pl.ds(3 * PREP + i0, BLK * 2)]
            va = gbuf[pl.ds(4 * PREP + i0, BLK * 2)]

            for u in range(BLK):
                cy1 = vy1[u]
                cx1 = vx1[u]
                cy2 = vy2[u]
                cx2 = vx2[u]
                ca = va[u]

                npairs = (nsel + 2 * LANES - 1) // (2 * LANES)

                def chk(j, found, cy1=cy1, cx1=cx1, cy2=cy2, cx2=cx2, ca=ca):
                    for v in range(2):
                        base = j * 2 * LANES + v * LANES
                        sy1 = sel[pl.ds(base, LANES)]
                        sx1 = sel[pl.ds(SELPAD + base, LANES)]
                        sy2 = sel[pl.ds(2 * SELPAD + base, LANES)]
                        sx2 = sel[pl.ds(3 * SELPAD + base, LANES)]
                        sa = sel[pl.ds(4 * SELPAD + base, LANES)]
                        yy1 = jnp.maximum(sy1, cy1)
                        xx1 = jnp.maximum(sx1, cx1)
                        yy2 = jnp.minimum(sy2, cy2)
                        xx2 = jnp.minimum(sx2, cx2)
                        inter = jnp.maximum(yy2 - yy1, _f32(0.0)) * \
                            jnp.maximum(xx2 - xx1, _f32(0.0))
                        denom = jnp.maximum(sa + ca - inter, eps)
                        bad = inter >= thr * denom
                        found = found + \
                            jnp.sum(jnp.where(bad, 1, 0).astype(_i32))
                    return found
                found = lax.fori_loop(0, npairs, chk, jnp.int32(0))

                keep = jnp.logical_and(found == 0, nsel < NOUT)

                @pl.when(keep)
                def _(cy1=cy1, cx1=cx1, cy2=cy2, cx2=cx2, ca=ca, nsel=nsel):
                    lv = lanes
                    vals5 = jnp.where(
                        lv == 0, cy1,
                        jnp.where(lv == 1, cx1,
                                  jnp.where(lv == 2, cy2,
                                            jnp.where(lv == 3, cx2, ca))))
                    sel_idx = lv * SELPAD + nsel
                    plsc.store_scatter(sel, [sel_idx], vals5, mask=lv < 5)
                    out_idx = 4 * nsel + lv
                    plsc.store_scatter(outbuf, [out_idx], vals5, mask=lv < 4)

                nsel = jnp.where(keep, nsel + 1, nsel)

            return bi + 1, nsel

        for d in descs[:8]:
            d.wait()
        boxes_half(0)
        st = lax.while_loop(nms_cond_blk(HHALF), nms_blk_body,
                            (jnp.int32(0), jnp.int32(0)))
        for d in descs[8:]:
            d.wait()
        boxes_half(1)
        lax.while_loop(nms_cond_blk(PRE), nms_blk_body, st)

        pltpu.sync_copy(outbuf, out_hbm.at[b])


@functools.partial(jax.jit, static_argnums=())
def kernel(rpn_probs, rpn_bbox, anchors):
    scores = rpn_probs[:, :, 1].reshape(-1)
    anch_p = anchors.transpose(0, 2, 1)
    delt_p = rpn_bbox.transpose(0, 2, 1)
    planes = jnp.concatenate([anch_p, delt_p], axis=1).reshape(-1)

    mesh = plsc.VectorSubcoreMesh(core_axis_name="c", subcore_axis_name="s")
    out = pl.kernel(
        _sc_body,
        out_type=jax.ShapeDtypeStruct((BATCH, 4 * NOUT), _f32),
        mesh=mesh,
        compiler_params=pltpu.CompilerParams(needs_layout_passes=False),
        scratch_types=[
            pltpu.VMEM((NA,), _i32),
            pltpu.VMEM((NA,), _i32),
            pltpu.VMEM((NBINS * LANES,), _i32),
            pltpu.VMEM((HROWS * GCH,), _i32),
            pltpu.VMEM((HROWS * GCH,), _i32),
            pltpu.VMEM((8 * PREP,), _f32),
            pltpu.VMEM((5 * SELPAD,), _f32),
            pltpu.VMEM((4 * NOUT,), _f32),
            pltpu.SemaphoreType.DMA,
            pltpu.SemaphoreType.DMA,
        ],
    )(scores, planes)
    return out.reshape(BATCH, NOUT, 4)


# speedup vs baseline: 2.2774x; 2.2774x over previous
import functools

import jax
import jax.numpy as jnp
from jax import lax
from jax.experimental import pallas as pl
from jax.experimental.pallas import tpu as pltpu
from jax.experimental.pallas import tpu_sc as plsc

BATCH = 8
NA = 20000
PRE = 6000
NOUT = 1000
THR = 0.7
LANES = 16
LCHUNK = NA // LANES
UNROLL = 5
NBINS = 1024
KMAX = 0x3F7FFFFF
GCH = 128
NGC = 48
PREP = NGC * GCH
HROWS = 24
SELPAD = 1024

_f32 = jnp.float32
_i32 = jnp.int32


def _sc_body(scores_hbm, planes_hbm, out_hbm,
             ord_a, ord_b, cnt_v, idx0, idx1, gbuf,
             sel, outbuf, sem0, sem1):
    wid = lax.axis_index("s") * 2 + lax.axis_index("c")

    @pl.when(wid < BATCH)
    def _():
        b = wid
        lanes = lax.iota(_i32, LANES)

        pltpu.sync_copy(scores_hbm.at[pl.ds(b * NA, NA)], gbuf.at[pl.ds(0, NA)])

        def key_of(sv):
            bits = plsc.bitcast(sv, _i32)
            return KMAX - bits

        def do_pass(p, ord_src, ord_dst, last):
            shift = 10 * p

            zv16 = jnp.zeros((LANES,), _i32)

            def zero_body(t, _):
                for u in range(4):
                    cnt_v[pl.ds((t * 4 + u) * LANES, LANES)] = zv16
                return 0
            lax.fori_loop(0, NBINS // 4, zero_body, 0)

            def elem_digit(t):
                pos = lanes * LCHUNK + t
                if ord_src is None:
                    ev = pos
                    kp = key_of(plsc.load_gather(gbuf, [ev]))
                    digit = kp & (NBINS - 1)
                else:
                    ov = plsc.load_gather(ord_src, [pos])
                    digit = lax.shift_right_logical(ov, 15) & (NBINS - 1)
                    ev = ov & 0x7FFF
                    kp = None
                return ev, digit, kp

            ones = jnp.ones((LANES,), _i32)

            def hist_body(t, _):
                for u in range(UNROLL):
                    _, digit, _ = elem_digit(t * UNROLL + u)
                    plsc.addupdate_scatter(cnt_v, [digit * LANES + lanes],
                                           ones)
                return 0
            lax.fori_loop(0, LCHUNK // UNROLL, hist_body, 0)

            def scan_body(t, carry):
                for u in range(4):
                    sl = pl.ds((t * 4 + u) * LANES, LANES)
                    v = cnt_v[sl]
                    incl = plsc.cumsum(v)
                    tot = jnp.sum(v)
                    cnt_v[sl] = incl - v + carry
                    carry = carry + tot
                return carry
            lax.fori_loop(0, NBINS // 4, scan_body, jnp.int32(0))

            def perm_body(t, _):
                for u in range(UNROLL):
                    ev, digit, kp = elem_digit(t * UNROLL + u)
                    slot = digit * LANES + lanes
                    pos = plsc.load_gather(cnt_v, [slot])
                    if last:
                        val = ev
                    else:
                        if kp is None:
                            kp = key_of(plsc.load_gather(gbuf, [ev]))
                        nd = lax.shift_right_logical(kp, shift + 10) \
                            & (NBINS - 1)
                        val = ev | lax.shift_left(nd, 15)
                    plsc.store_scatter(ord_dst, [pos], val)
                    plsc.addupdate_scatter(cnt_v, [slot], ones)
                return 0
            lax.fori_loop(0, LCHUNK // UNROLL, perm_body, 0)

        do_pass(0, None, ord_b, False)
        do_pass(1, ord_b, ord_a, False)
        do_pass(2, ord_a, ord_b, True)

        HHALF = HROWS * GCH

        def fill_rel(ib, h):
            def fb(t, _):
                ib[pl.ds(t * LANES, LANES)] = \
                    ord_b[pl.ds(h * HHALF + t * LANES, LANES)]
                return 0
            lax.fori_loop(0, HHALF // LANES, fb, 0)

        descs = []
        for h, ib, sem in ((0, idx0, sem0), (1, idx1, sem1)):
            fill_rel(ib, h)
            for j in range(8):
                d = pltpu.make_async_copy(
                    planes_hbm.at[pl.ds((b * 8 + j) * NA, NA)].at[ib],
                    gbuf.at[pl.ds(j * PREP + h * HHALF, HHALF)],
                    sem)
                d.start()
                descs.append(d)

        def boxes_half(h):
            def boxes_t(t, _):
                sl = pl.ds(h * HHALF + t * LANES, LANES)
                ay1 = gbuf[pl.ds(0 * PREP + h * HHALF + t * LANES, LANES)]
                ax1 = gbuf[pl.ds(1 * PREP + h * HHALF + t * LANES, LANES)]
                ay2 = gbuf[pl.ds(2 * PREP + h * HHALF + t * LANES, LANES)]
                ax2 = gbuf[pl.ds(3 * PREP + h * HHALF + t * LANES, LANES)]
                d0 = gbuf[pl.ds(4 * PREP + h * HHALF + t * LANES, LANES)] \
                    * _f32(0.1)
                d1 = gbuf[pl.ds(5 * PREP + h * HHALF + t * LANES, LANES)] \
                    * _f32(0.1)
                d2 = gbuf[pl.ds(6 * PREP + h * HHALF + t * LANES, LANES)] \
                    * _f32(0.2)
                d3 = gbuf[pl.ds(7 * PREP + h * HHALF + t * LANES, LANES)] \
                    * _f32(0.2)
                hh = ay2 - ay1
                w = ax2 - ax1
                cy = ay1 + _f32(0.5) * hh + d0 * hh
                cx = ax1 + _f32(0.5) * w + d1 * w
                h2 = hh * jnp.exp(d2)
                w2 = w * jnp.exp(d3)
                y1 = cy - _f32(0.5) * h2
                x1 = cx - _f32(0.5) * w2
                y2 = y1 + h2
                x2 = x1 + w2
                one = _f32(1.0)
                zero = _f32(0.0)
                y1 = jnp.minimum(jnp.maximum(y1, zero), one)
                x1 = jnp.minimum(jnp.maximum(x1, zero), one)
                y2 = jnp.minimum(jnp.maximum(y2, zero), one)
                x2 = jnp.minimum(jnp.maximum(x2, zero), one)
                gbuf[pl.ds(0 * PREP + h * HHALF + t * LANES, LANES)] = y1
                gbuf[pl.ds(1 * PREP + h * HHALF + t * LANES, LANES)] = x1
                gbuf[pl.ds(2 * PREP + h * HHALF + t * LANES, LANES)] = y2
                gbuf[pl.ds(3 * PREP + h * HHALF + t * LANES, LANES)] = x2
                gbuf[pl.ds(4 * PREP + h * HHALF + t * LANES, LANES)] = \
                    (y2 - y1) * (x2 - x1)
                return 0
            lax.fori_loop(0, HHALF // LANES, boxes_t, 0)

        def sent_body(t, _):
            two = jnp.full((LANES,), 2.0, _f32)
            zv = jnp.zeros((LANES,), _f32)
            for r in range(4):
                sel[pl.ds(r * SELPAD + t * LANES, LANES)] = two
            sel[pl.ds(4 * SELPAD + t * LANES, LANES)] = zv
            return 0
        lax.fori_loop(0, SELPAD // LANES, sent_body, 0)

        def zout_body(t, _):
            outbuf[pl.ds(t * LANES, LANES)] = jnp.zeros((LANES,), _f32)
            return 0
        lax.fori_loop(0, (4 * NOUT) // LANES, zout_body, 0)

        thr = _f32(THR)
        eps = _f32(1e-12)

        BLK = 8

        def nms_cond_blk(limit):
            def cond(state):
                bi, nsel = state
                return jnp.logical_and(bi < limit // BLK, nsel < NOUT)
            return cond

        def nms_blk_body(state):
            bi, nsel = state
            i0 = bi * BLK
            vy1 = gbuf[pl.ds(0 * PREP + i0, BLK * 2)]
            vx1 = gbuf[pl.ds(1 * PREP + i0, BLK * 2)]
            vy2 = gbuf[pl.ds(2 * PREP + i0, BLK * 2)]
            vx2 = gbuf[pl.ds(3 * PREP + i0, BLK * 2)]
            va = gbuf[pl.ds(4 * PREP + i0, BLK * 2)]

            for u in range(BLK):
                cy1 = vy1[u]
                cx1 = vx1[u]
                cy2 = vy2[u]
                cx2 = vx2[u]
                ca = va[u]

                npairs = (nsel + 2 * LANES - 1) // (2 * LANES)

                def chk(j, found, cy1=cy1, cx1=cx1, cy2=cy2, cx2=cx2, ca=ca):
                    for v in range(2):
                        base = j * 2 * LANES + v * LANES
                        sy1 = sel[pl.ds(base, LANES)]
                        sx1 = sel[pl.ds(SELPAD + base, LANES)]
                        sy2 = sel[pl.ds(2 * SELPAD + base, LANES)]
                        sx2 = sel[pl.ds(3 * SELPAD + base, LANES)]
                        sa = sel[pl.ds(4 * SELPAD + base, LANES)]
                        yy1 = jnp.maximum(sy1, cy1)
                        xx1 = jnp.maximum(sx1, cx1)
                        yy2 = jnp.minimum(sy2, cy2)
                        xx2 = jnp.minimum(sx2, cx2)
                        inter = jnp.maximum(yy2 - yy1, _f32(0.0)) * \
                            jnp.maximum(xx2 - xx1, _f32(0.0))
                        denom = jnp.maximum(sa + ca - inter, eps)
                        bad = inter >= thr * denom
                        found = found + \
                            jnp.sum(jnp.where(bad, 1, 0).astype(_i32))
                    return found
                found = lax.fori_loop(0, npairs, chk, jnp.int32(0))

                keep = jnp.logical_and(found == 0, nsel < NOUT)

                @pl.when(keep)
                def _(cy1=cy1, cx1=cx1, cy2=cy2, cx2=cx2, ca=ca, nsel=nsel):
                    lv = lanes
                    vals5 = jnp.where(
                        lv == 0, cy1,
                        jnp.where(lv == 1, cx1,
                                  jnp.where(lv == 2, cy2,
                                            jnp.where(lv == 3, cx2, ca))))
                    sel_idx = lv * SELPAD + nsel
                    plsc.store_scatter(sel, [sel_idx], vals5, mask=lv < 5)
                    out_idx = 4 * nsel + lv
                    plsc.store_scatter(outbuf, [out_idx], vals5, mask=lv < 4)

                nsel = jnp.where(keep, nsel + 1, nsel)

            return bi + 1, nsel

        for d in descs[:8]:
            d.wait()
        boxes_half(0)
        st = lax.while_loop(nms_cond_blk(HHALF), nms_blk_body,
                            (jnp.int32(0), jnp.int32(0)))
        for d in descs[8:]:
            d.wait()
        boxes_half(1)
        lax.while_loop(nms_cond_blk(PRE), nms_blk_body, st)

        pltpu.sync_copy(outbuf, out_hbm.at[b])


@functools.partial(jax.jit, static_argnums=())
def kernel(rpn_probs, rpn_bbox, anchors):
    scores = rpn_probs[:, :, 1].reshape(-1)
    anch_p = anchors.transpose(0, 2, 1)
    delt_p = rpn_bbox.transpose(0, 2, 1)
    planes = jnp.concatenate([anch_p, delt_p], axis=1).reshape(-1)

    mesh = plsc.VectorSubcoreMesh(core_axis_name="c", subcore_axis_name="s")
    out = pl.kernel(
        _sc_body,
        out_type=jax.ShapeDtypeStruct((BATCH, 4 * NOUT), _f32),
        mesh=mesh,
        compiler_params=pltpu.CompilerParams(needs_layout_passes=False),
        scratch_types=[
            pltpu.VMEM((NA,), _i32),
            pltpu.VMEM((NA,), _i32),
            pltpu.VMEM((NBINS * LANES,), _i32),
            pltpu.VMEM((HROWS * GCH,), _i32),
            pltpu.VMEM((HROWS * GCH,), _i32),
            pltpu.VMEM((8 * PREP,), _f32),
            pltpu.VMEM((5 * SELPAD,), _f32),
            pltpu.VMEM((4 * NOUT,), _f32),
            pltpu.SemaphoreType.DMA,
            pltpu.SemaphoreType.DMA,
        ],
    )(scores, planes)
    return out.reshape(BATCH, NOUT, 4)
